# baseline (speedup 1.0000x reference)
import functools

import jax
import jax.numpy as jnp
from jax import lax
from jax.experimental import pallas as pl
from jax.experimental.pallas import tpu as pltpu
from jax.experimental.pallas import tpu_sc as plsc

N = 50000
D = 256
S = 6

NW = 32
NPAD = 50176
CH = NPAD // NW
SZ = 224
NSUB = CH // SZ

BN = 2000
_SC_MESH = plsc.VectorSubcoreMesh(core_axis_name="c", subcore_axis_name="s")


def _zero_buf(buf_ref):
    zeros = jnp.zeros((16,), jnp.float32)

    def body(i, _):
        base = i * 256
        for k in range(16):
            buf_ref[pl.ds(base + k * 16, 16)] = zeros
        return 0

    lax.fori_loop(0, SZ * D // 256, body, 0)


def _scatter_chunk(nb_ref, buf_ref):
    ones = jnp.ones((16,), jnp.float32)
    lane = lax.iota(jnp.int32, 16)

    def body(g, _):
        node0 = g * 16
        gather_base = node0 * S + lane * S
        row_base = (node0 + lane) * D
        for s in range(S):
            ids = plsc.load_gather(nb_ref, [gather_base + s])
            plsc.addupdate_scatter(buf_ref, [row_base + ids], ones)
        return 0

    lax.fori_loop(0, SZ // 16, body, 0)


@functools.partial(
    pl.kernel,
    mesh=_SC_MESH,
    compiler_params=pltpu.CompilerParams(needs_layout_passes=False),
    out_type=jax.ShapeDtypeStruct((NPAD * D,), jnp.float32),
    scratch_types=[
        pltpu.VMEM((SZ * S,), jnp.int32),
        pltpu.VMEM((SZ * S,), jnp.int32),
        pltpu.VMEM((SZ * D,), jnp.float32),
        pltpu.VMEM((SZ * D,), jnp.float32),
        pltpu.SemaphoreType.DMA,
        pltpu.SemaphoreType.DMA,
    ],
)
def _sc_counts(nb_hbm, out_hbm, nb_a, nb_b, buf_a, buf_b, sem_a, sem_b):
    wid = lax.axis_index("s") * 2 + lax.axis_index("c")
    base = wid * CH
    handles = [None] * NSUB
    for c in range(NSUB):
        nb_v = (nb_a, nb_b)[c % 2]
        buf_v = (buf_a, buf_b)[c % 2]
        sem = (sem_a, sem_b)[c % 2]
        row0 = base + c * SZ
        if c >= 2:
            handles[c - 2].wait()
        pltpu.sync_copy(nb_hbm.at[pl.ds(row0 * S, SZ * S)], nb_v)
        _zero_buf(buf_v)
        _scatter_chunk(nb_v, buf_v)
        handles[c] = pltpu.async_copy(
            buf_v, out_hbm.at[pl.ds(row0 * D, SZ * D)], sem)
    handles[NSUB - 2].wait()
    handles[NSUB - 1].wait()


def _project_tables_kernel(emb_ref, w0_ref, w1_ref, t0_ref, t1_ref,
                           w0a_ref, w1a_ref):
    emb = emb_ref[...]
    scale = 1.0 / S
    for w_ref, t_ref, wa_ref in ((w0_ref, t0_ref, w0a_ref),
                                 (w1_ref, t1_ref, w1a_ref)):
        wb = w_ref[:, D:]
        t = jax.lax.dot_general(
            emb, wb, (((1,), (1,)), ((), ())),
            preferred_element_type=jnp.float32,
            precision=jax.lax.Precision.HIGHEST,
        )
        t_ref[...] = (t * scale).astype(jnp.bfloat16)
        wa_ref[...] = w_ref[:, :D].astype(jnp.bfloat16)


def _sage_mm_kernel(x0_ref, cnt_ref, w0a_ref, w1a_ref, t0_ref, t1_ref, out_ref):
    counts = cnt_ref[...].astype(jnp.bfloat16)
    emb = x0_ref[...].astype(jnp.bfloat16)
    for layer, (wa_ref, t_ref) in enumerate(((w0a_ref, t0_ref),
                                             (w1a_ref, t1_ref))):
        h = jax.lax.dot_general(
            emb, wa_ref[...], (((1,), (1,)), ((), ())),
            preferred_element_type=jnp.float32,
        )
        h = h + jax.lax.dot_general(
            counts, t_ref[...], (((1,), (0,)), ((), ())),
            preferred_element_type=jnp.float32,
        )
        h = jnp.maximum(h, 0.0)
        if layer == 0:
            emb = h.astype(jnp.bfloat16)
    out_ref[...] = h


@jax.jit
def kernel(x0, emb_table, W0, W1, neighbors):
    nb = neighbors.astype(jnp.int32)
    nb_flat = jnp.pad(nb, ((0, NPAD - N), (0, 0))).reshape(-1)
    counts_flat = _sc_counts(nb_flat)
    counts = counts_flat.reshape(NPAD, D)

    t0, t1, w0a, w1a = pl.pallas_call(
        _project_tables_kernel,
        out_shape=(
            jax.ShapeDtypeStruct((D, D), jnp.bfloat16),
            jax.ShapeDtypeStruct((D, D), jnp.bfloat16),
            jax.ShapeDtypeStruct((D, D), jnp.bfloat16),
            jax.ShapeDtypeStruct((D, D), jnp.bfloat16),
        ),
    )(emb_table, W0, W1)

    grid = N // BN
    small = pl.BlockSpec((D, D), lambda i: (0, 0))
    out = pl.pallas_call(
        _sage_mm_kernel,
        grid=(grid,),
        in_specs=[
            pl.BlockSpec((BN, D), lambda i: (i, 0)),
            pl.BlockSpec((BN, D), lambda i: (i, 0)),
            small, small, small, small,
        ],
        out_specs=pl.BlockSpec((BN, D), lambda i: (i, 0)),
        out_shape=jax.ShapeDtypeStruct((N, D), jnp.float32),
    )(x0, counts, w0a, w1a, t0, t1)
    return out


# speedup vs baseline: 2.4719x; 2.4719x over previous
import functools

import jax
import jax.numpy as jnp
from jax.experimental import pallas as pl

N = 50000
D = 256
S = 6
BN = 2000
SUB = 10
BS = BN // SUB


def _project_tables_kernel(emb_ref, w0_ref, w1_ref, t0_ref, t1_ref,
                           w0a_ref, w1a_ref):
    emb = emb_ref[...]
    scale = 1.0 / S
    for w_ref, t_ref, wa_ref in ((w0_ref, t0_ref, w0a_ref),
                                 (w1_ref, t1_ref, w1a_ref)):
        wb = w_ref[:, D:]
        t = jax.lax.dot_general(
            emb, wb, (((1,), (1,)), ((), ())),
            preferred_element_type=jnp.float32,
            precision=jax.lax.Precision.HIGHEST,
        )
        t_ref[...] = (t * scale).astype(jnp.bfloat16)
        wa_ref[...] = w_ref[:, :D].astype(jnp.bfloat16)


def _sage_kernel(x0_ref, nb_ref, w0a_ref, w1a_ref, t0_ref, t1_ref, out_ref):
    col_ids = jax.lax.broadcasted_iota(jnp.int32, (BS, D), 1)
    w0a = w0a_ref[...]
    w1a = w1a_ref[...]
    t0 = t0_ref[...]
    t1 = t1_ref[...]
    for b in range(SUB):
        rows = pl.ds(b * BS, BS)
        nb = nb_ref[rows, :]
        counts = jnp.zeros((BS, D), jnp.float32)
        for s in range(S):
            counts = counts + (nb[:, s][:, None] == col_ids).astype(jnp.float32)
        counts = counts.astype(jnp.bfloat16)

        emb = x0_ref[rows, :].astype(jnp.bfloat16)
        for layer, (wa, t) in enumerate(((w0a, t0), (w1a, t1))):
            h = jax.lax.dot_general(
                emb, wa, (((1,), (1,)), ((), ())),
                preferred_element_type=jnp.float32,
            )
            h = h + jax.lax.dot_general(
                counts, t, (((1,), (0,)), ((), ())),
                preferred_element_type=jnp.float32,
            )
            h = jnp.maximum(h, 0.0)
            if layer == 0:
                emb = h.astype(jnp.bfloat16)
        out_ref[rows, :] = h


@jax.jit
def kernel(x0, emb_table, W0, W1, neighbors):
    nb = neighbors.astype(jnp.int32)
    t0, t1, w0a, w1a = pl.pallas_call(
        _project_tables_kernel,
        out_shape=(
            jax.ShapeDtypeStruct((D, D), jnp.bfloat16),
            jax.ShapeDtypeStruct((D, D), jnp.bfloat16),
            jax.ShapeDtypeStruct((D, D), jnp.bfloat16),
            jax.ShapeDtypeStruct((D, D), jnp.bfloat16),
        ),
    )(emb_table, W0, W1)

    grid = N // BN
    small = pl.BlockSpec((D, D), lambda i: (0, 0))
    out = pl.pallas_call(
        _sage_kernel,
        grid=(grid,),
        in_specs=[
            pl.BlockSpec((BN, D), lambda i: (i, 0)),
            pl.BlockSpec((BN, S), lambda i: (i, 0)),
            small, small, small, small,
        ],
        out_specs=pl.BlockSpec((BN, D), lambda i: (i, 0)),
        out_shape=jax.ShapeDtypeStruct((N, D), jnp.float32),
    )(x0, nb, w0a, w1a, t0, t1)
    return out


# speedup vs baseline: 2.6388x; 1.0675x over previous
import jax
import jax.numpy as jnp
from jax.experimental import pallas as pl

N = 50000
D = 256
S = 6
BN = 10000
SUB = 50
BS = BN // SUB


def _project_tables_kernel(emb_ref, w0_ref, w1_ref, t0_ref, t1_ref,
                           w0a_ref, w1a_ref):
    emb = emb_ref[...]
    scale = 1.0 / S
    for w_ref, t_ref, wa_ref in ((w0_ref, t0_ref, w0a_ref),
                                 (w1_ref, t1_ref, w1a_ref)):
        wb = w_ref[:, D:]
        t = jax.lax.dot_general(
            emb, wb, (((1,), (1,)), ((), ())),
            preferred_element_type=jnp.float32,
            precision=jax.lax.Precision.HIGHEST,
        )
        t_ref[...] = (t * scale).astype(jnp.bfloat16)
        wa_ref[...] = w_ref[:, :D].astype(jnp.bfloat16)


def _sage_kernel(x0_ref, nb_ref, w0a_ref, w1a_ref, t0_ref, t1_ref, out_ref):
    col_ids = jax.lax.broadcasted_iota(jnp.int32, (BS, D), 1)
    w0a = w0a_ref[...]
    w1a = w1a_ref[...]
    t0 = t0_ref[...]
    t1 = t1_ref[...]
    for b in range(SUB):
        rows = pl.ds(b * BS, BS)
        nb = nb_ref[rows, :]
        counts = jnp.zeros((BS, D), jnp.float32)
        for s in range(S):
            counts = counts + (nb[:, s][:, None] == col_ids).astype(jnp.float32)
        counts = counts.astype(jnp.bfloat16)

        emb = x0_ref[rows, :].astype(jnp.bfloat16)
        for layer, (wa, t) in enumerate(((w0a, t0), (w1a, t1))):
            h = jax.lax.dot_general(
                emb, wa, (((1,), (1,)), ((), ())),
                preferred_element_type=jnp.float32,
            )
            h = h + jax.lax.dot_general(
                counts, t, (((1,), (0,)), ((), ())),
                preferred_element_type=jnp.float32,
            )
            h = jnp.maximum(h, 0.0)
            if layer == 0:
                emb = h.astype(jnp.bfloat16)
        out_ref[rows, :] = h


@jax.jit
def kernel(x0, emb_table, W0, W1, neighbors):
    nb = neighbors.astype(jnp.int32)
    t0, t1, w0a, w1a = pl.pallas_call(
        _project_tables_kernel,
        out_shape=(
            jax.ShapeDtypeStruct((D, D), jnp.bfloat16),
            jax.ShapeDtypeStruct((D, D), jnp.bfloat16),
            jax.ShapeDtypeStruct((D, D), jnp.bfloat16),
            jax.ShapeDtypeStruct((D, D), jnp.bfloat16),
        ),
    )(emb_table, W0, W1)

    grid = N // BN
    small = pl.BlockSpec((D, D), lambda i: (0, 0))
    out = pl.pallas_call(
        _sage_kernel,
        grid=(grid,),
        in_specs=[
            pl.BlockSpec((BN, D), lambda i: (i, 0)),
            pl.BlockSpec((BN, S), lambda i: (i, 0)),
            small, small, small, small,
        ],
        out_specs=pl.BlockSpec((BN, D), lambda i: (i, 0)),
        out_shape=jax.ShapeDtypeStruct((N, D), jnp.float32),
    )(x0, nb, w0a, w1a, t0, t1)
    return out
